# baseline (speedup 1.0000x reference)
import functools

import jax
import jax.numpy as jnp
from jax.experimental import pallas as pl
from jax.experimental.pallas import tpu as pltpu

N = 800
D = 64
F_IN = 128
TOP_K = 10
ALPHA = 0.3
RB = 160
BIG_IDX = 1 << 30


def _bdot(a, b):
    return jnp.dot(a.astype(jnp.bfloat16), b.astype(jnp.bfloat16),
                   preferred_element_type=jnp.float32)


def _prep_kernel(x_ref, encW_ref, encb_ref, lng_ref, lnb_ref,
                 w1a_ref, w1b_ref, b1_ref, msgpW_ref, msgpb_ref,
                 msgnW_ref, msgnb_ref,
                 h_out, sip_out, sj_out, msgp_out, msgn_out):
    x = x_ref[...]
    h0 = _bdot(x, encW_ref[...]) + encb_ref[...]
    mu = jnp.mean(h0, axis=1, keepdims=True)
    var = jnp.mean((h0 - mu) ** 2, axis=1, keepdims=True)
    h = jnp.maximum((h0 - mu) / jnp.sqrt(var + 1e-5) * lng_ref[...] + lnb_ref[...], 0.0)
    h_out[...] = h
    sip_out[...] = _bdot(h, w1a_ref[...]) + b1_ref[...]
    sj_out[...] = _bdot(h, w1b_ref[...])
    msgp_out[...] = _bdot(h, msgpW_ref[...]) + msgpb_ref[...]
    msgn_out[...] = _bdot(h, msgnW_ref[...]) + msgnb_ref[...]


def _gru(m, h, Wr, Wz, Wn, Ur, Uz, Un, br, bz, bin_, bhn):
    r = jax.nn.sigmoid(_bdot(m, Wr) + _bdot(h, Ur) + br)
    z = jax.nn.sigmoid(_bdot(m, Wz) + _bdot(h, Uz) + bz)
    n = jnp.tanh(_bdot(m, Wn) + bin_ + r * (_bdot(h, Un) + bhn))
    return (1.0 - z) * n + z * h


def _main_kernel(sip_ref, sjT_ref, w2_ref, b2_ref, mfg_ref, h_ref,
                 msgp_ref, msgn_ref,
                 pWr_ref, pWz_ref, pWn_ref, pUr_ref, pUz_ref, pUn_ref,
                 pbr_ref, pbz_ref, pbin_ref, pbhn_ref,
                 nWr_ref, nWz_ref, nWn_ref, nUr_ref, nUz_ref, nUn_ref,
                 nbr_ref, nbz_ref, nbin_ref, nbhn_ref,
                 combA_ref, combB_ref, combb_ref,
                 out_ref):
    i = pl.program_id(0)
    row0 = i * RB
    sip = sip_ref[...]
    sjT = sjT_ref[...]

    w2q = w2_ref[...].astype(jnp.bfloat16).astype(jnp.float32)
    acc = jnp.zeros((RB, N), dtype=jnp.float32)
    for d in range(D):
        col = sip[:, d:d + 1]
        row = sjT[d:d + 1, :]
        wd = w2q[d:d + 1, :]
        rp = jnp.maximum(col + row, 0.0).astype(jnp.bfloat16).astype(jnp.float32)
        acc = acc + rp * wd
    logits = acc + b2_ref[...]

    att = jax.nn.sigmoid(logits)
    jota = jax.lax.broadcasted_iota(jnp.int32, (RB, N), 1)
    riota = jax.lax.broadcasted_iota(jnp.int32, (RB, N), 0) + row0
    work = jnp.where(jota == riota, -1.0, att)

    mask = jnp.zeros((RB, N), dtype=jnp.float32)
    for _ in range(TOP_K):
        mx = jnp.max(work, axis=1, keepdims=True)
        cand = jnp.where(work >= mx, jota, BIG_IDX)
        amin = jnp.min(cand, axis=1, keepdims=True)
        sel = jota == amin
        mask = jnp.where(sel, 1.0, mask)
        work = jnp.where(sel, -1.0, work)

    att_f = att * mask
    mfg = mfg_ref[...]
    adj_p = att_f * (mfg > ALPHA).astype(jnp.float32)
    adj_p = adj_p / (jnp.sum(adj_p, axis=1, keepdims=True) + 1e-8)
    adj_n = att_f * (mfg < -ALPHA).astype(jnp.float32)
    adj_n = adj_n / (jnp.sum(adj_n, axis=1, keepdims=True) + 1e-8)

    m_pos = _bdot(adj_p, msgp_ref[...])
    m_neg = _bdot(adj_n, msgn_ref[...])

    h = h_ref[...]
    h_pos = _gru(m_pos, h, pWr_ref[...], pWz_ref[...], pWn_ref[...],
                 pUr_ref[...], pUz_ref[...], pUn_ref[...],
                 pbr_ref[...], pbz_ref[...], pbin_ref[...], pbhn_ref[...])
    h_neg = _gru(m_neg, h, nWr_ref[...], nWz_ref[...], nWn_ref[...],
                 nUr_ref[...], nUz_ref[...], nUn_ref[...],
                 nbr_ref[...], nbz_ref[...], nbin_ref[...], nbhn_ref[...])

    out_ref[...] = (_bdot(h_pos, combA_ref[...]) + _bdot(h_neg, combB_ref[...])
                    + combb_ref[...])


def _row2(v):
    return v.reshape(1, -1)


def kernel(x_risk, money_flow_graph, enc_W, enc_b, ln_g, ln_b, att_W1, att_b1, att_W2, att_b2,
           msg_pos_W, msg_pos_b, gru_pos_Wih, gru_pos_Whh, gru_pos_bih, gru_pos_bhh,
           msg_neg_W, msg_neg_b, gru_neg_Wih, gru_neg_Whh, gru_neg_bih, gru_neg_bhh,
           comb_W, comb_b):
    x = x_risk[0, -1]
    mfg = money_flow_graph[0]

    f32 = jnp.float32
    prep_out = pl.pallas_call(
        _prep_kernel,
        out_shape=[
            jax.ShapeDtypeStruct((N, D), f32),
            jax.ShapeDtypeStruct((N, D), f32),
            jax.ShapeDtypeStruct((N, D), f32),
            jax.ShapeDtypeStruct((N, D), f32),
            jax.ShapeDtypeStruct((N, D), f32),
        ],
    )(x, enc_W, _row2(enc_b), _row2(ln_g), _row2(ln_b),
      att_W1[:D], att_W1[D:], _row2(att_b1), msg_pos_W, _row2(msg_pos_b),
      msg_neg_W, _row2(msg_neg_b))
    h, sip, sj, msgp, msgn = prep_out
    sjT = sj.T

    def gates(Wih, Whh, bih, bhh):
        Wr, Wz, Wn = (Wih[:D].T, Wih[D:2 * D].T, Wih[2 * D:].T)
        Ur, Uz, Un = (Whh[:D].T, Whh[D:2 * D].T, Whh[2 * D:].T)
        br = _row2(bih[:D] + bhh[:D])
        bz = _row2(bih[D:2 * D] + bhh[D:2 * D])
        bin_ = _row2(bih[2 * D:])
        bhn = _row2(bhh[2 * D:])
        return Wr, Wz, Wn, Ur, Uz, Un, br, bz, bin_, bhn

    pos_g = gates(gru_pos_Wih, gru_pos_Whh, gru_pos_bih, gru_pos_bhh)
    neg_g = gates(gru_neg_Wih, gru_neg_Whh, gru_neg_bih, gru_neg_bhh)

    blk = lambda r, c: pl.BlockSpec((r, c), lambda i: (i, 0))
    full = lambda r, c: pl.BlockSpec((r, c), lambda i: (0, 0))

    grid = N // RB
    in_specs = [
        blk(RB, D),
        full(D, N),
        full(D, 1),
        full(1, 1),
        blk(RB, N),
        blk(RB, D),
        full(N, D),
        full(N, D),
    ]
    in_specs += [full(D, D)] * 6 + [full(1, D)] * 4
    in_specs += [full(D, D)] * 6 + [full(1, D)] * 4
    in_specs += [full(D, D), full(D, D), full(1, D)]

    out = pl.pallas_call(
        _main_kernel,
        grid=(grid,),
        in_specs=in_specs,
        out_specs=blk(RB, D),
        out_shape=jax.ShapeDtypeStruct((N, D), f32),
        compiler_params=pltpu.CompilerParams(
            dimension_semantics=("parallel",)),
    )(sip, sjT, att_W2, att_b2.reshape(1, 1), mfg, h, msgp, msgn,
      *pos_g, *neg_g, comb_W[:D], comb_W[D:], _row2(comb_b))

    return out[None]


# speedup vs baseline: 1.0860x; 1.0860x over previous
import jax
import jax.numpy as jnp
from jax.experimental import pallas as pl
from jax.experimental.pallas import tpu as pltpu

N = 800
D = 64
F_IN = 128
T_IN = 16
TOP_K = 10
ALPHA = 0.3
RB = 160
BIG_IDX = 1 << 30


def _bdot(a, b, dn=None):
    a = a.astype(jnp.bfloat16).astype(jnp.float32)
    b = b.astype(jnp.bfloat16).astype(jnp.float32)
    if dn is None:
        return jnp.dot(a, b, preferred_element_type=jnp.float32)
    return jax.lax.dot_general(a, b, dn, preferred_element_type=jnp.float32)


_DN_NT = (((1,), (1,)), ((), ()))


def _gru(m, h, Wih, Whh, bih, bhh):
    ir = _bdot(m, Wih[0:D], _DN_NT)
    iz = _bdot(m, Wih[D:2 * D], _DN_NT)
    inn = _bdot(m, Wih[2 * D:], _DN_NT)
    hr = _bdot(h, Whh[0:D], _DN_NT)
    hz = _bdot(h, Whh[D:2 * D], _DN_NT)
    hn = _bdot(h, Whh[2 * D:], _DN_NT)
    r = jax.nn.sigmoid(ir + hr + bih[:, 0:D] + bhh[:, 0:D])
    z = jax.nn.sigmoid(iz + hz + bih[:, D:2 * D] + bhh[:, D:2 * D])
    n = jnp.tanh(inn + bih[:, 2 * D:] + r * (hn + bhh[:, 2 * D:]))
    return (1.0 - z) * n + z * h


def _kernel(x_ref, mfg_ref, encW_ref, encb_ref, lng_ref, lnb_ref,
            w1_ref, b1_ref, w2_ref, b2_ref,
            msgpW_ref, msgpb_ref, pWih_ref, pWhh_ref, pbih_ref, pbhh_ref,
            msgnW_ref, msgnb_ref, nWih_ref, nWhh_ref, nbih_ref, nbhh_ref,
            combW_ref, combb_ref,
            out_ref,
            h_s, sip_s, sjT_s, msgp_s, msgn_s):
    i = pl.program_id(0)
    row0 = i * RB

    @pl.when(i == 0)
    def _prep():
        x = x_ref[0, 0]
        h0 = _bdot(x, encW_ref[...]) + encb_ref[...]
        mu = jnp.mean(h0, axis=1, keepdims=True)
        var = jnp.mean((h0 - mu) ** 2, axis=1, keepdims=True)
        h = jnp.maximum((h0 - mu) / jnp.sqrt(var + 1e-5) * lng_ref[...]
                        + lnb_ref[...], 0.0)
        h_s[...] = h
        sip_s[...] = _bdot(h, w1_ref[0:D]) + b1_ref[...]
        sjT_s[...] = jnp.swapaxes(_bdot(h, w1_ref[D:]), 0, 1)
        msgp_s[...] = _bdot(h, msgpW_ref[...]) + msgpb_ref[...]
        msgn_s[...] = _bdot(h, msgnW_ref[...]) + msgnb_ref[...]

    sip = sip_s[pl.ds(row0, RB), :]
    sjT = sjT_s[...]

    w2q = w2_ref[...].astype(jnp.bfloat16).astype(jnp.float32)
    acc = jnp.zeros((RB, N), dtype=jnp.float32)
    for d in range(D):
        col = sip[:, d:d + 1]
        row = sjT[d:d + 1, :]
        wd = w2q[d:d + 1, :]
        rp = jnp.maximum(col + row, 0.0).astype(jnp.bfloat16).astype(jnp.float32)
        acc = acc + rp * wd
    logits = acc + b2_ref[...]

    att = jax.nn.sigmoid(logits)
    jota = jax.lax.broadcasted_iota(jnp.int32, (RB, N), 1)
    riota = jax.lax.broadcasted_iota(jnp.int32, (RB, N), 0) + row0
    work = jnp.where(jota == riota, -1.0, att)

    mask = jnp.zeros((RB, N), dtype=jnp.float32)
    for _ in range(TOP_K):
        mx = jnp.max(work, axis=1, keepdims=True)
        cand = jnp.where(work >= mx, jota, BIG_IDX)
        amin = jnp.min(cand, axis=1, keepdims=True)
        sel = jota == amin
        mask = jnp.where(sel, 1.0, mask)
        work = jnp.where(sel, -1.0, work)

    att_f = att * mask
    mfg = mfg_ref[0]
    adj_p = att_f * (mfg > ALPHA).astype(jnp.float32)
    adj_p = adj_p / (jnp.sum(adj_p, axis=1, keepdims=True) + 1e-8)
    adj_n = att_f * (mfg < -ALPHA).astype(jnp.float32)
    adj_n = adj_n / (jnp.sum(adj_n, axis=1, keepdims=True) + 1e-8)

    m_pos = _bdot(adj_p, msgp_s[...])
    m_neg = _bdot(adj_n, msgn_s[...])

    h = h_s[pl.ds(row0, RB), :]
    h_pos = _gru(m_pos, h, pWih_ref[...], pWhh_ref[...], pbih_ref[...], pbhh_ref[...])
    h_neg = _gru(m_neg, h, nWih_ref[...], nWhh_ref[...], nbih_ref[...], nbhh_ref[...])

    out_ref[...] = (_bdot(h_pos, combW_ref[0:D]) + _bdot(h_neg, combW_ref[D:])
                    + combb_ref[...])


def _row2(v):
    return v.reshape(1, -1)


def kernel(x_risk, money_flow_graph, enc_W, enc_b, ln_g, ln_b, att_W1, att_b1, att_W2, att_b2,
           msg_pos_W, msg_pos_b, gru_pos_Wih, gru_pos_Whh, gru_pos_bih, gru_pos_bhh,
           msg_neg_W, msg_neg_b, gru_neg_Wih, gru_neg_Whh, gru_neg_bih, gru_neg_bhh,
           comb_W, comb_b):
    f32 = jnp.float32
    grid = N // RB

    full = lambda shape: pl.BlockSpec(shape, lambda i: (0,) * len(shape))
    in_specs = [
        pl.BlockSpec((1, 1, N, F_IN), lambda i: (0, T_IN - 1, 0, 0)),
        pl.BlockSpec((1, RB, N), lambda i: (0, i, 0)),
        full((F_IN, D)),
        full((1, D)),
        full((1, D)),
        full((1, D)),
        full((2 * D, D)),
        full((1, D)),
        full((D, 1)),
        full((1, 1)),
        full((D, D)),
        full((1, D)),
        full((3 * D, D)),
        full((3 * D, D)),
        full((1, 3 * D)),
        full((1, 3 * D)),
        full((D, D)),
        full((1, D)),
        full((3 * D, D)),
        full((3 * D, D)),
        full((1, 3 * D)),
        full((1, 3 * D)),
        full((2 * D, D)),
        full((1, D)),
    ]

    out = pl.pallas_call(
        _kernel,
        grid=(grid,),
        in_specs=in_specs,
        out_specs=pl.BlockSpec((RB, D), lambda i: (i, 0)),
        out_shape=jax.ShapeDtypeStruct((N, D), f32),
        scratch_shapes=[
            pltpu.VMEM((N, D), f32),
            pltpu.VMEM((N, D), f32),
            pltpu.VMEM((D, N), f32),
            pltpu.VMEM((N, D), f32),
            pltpu.VMEM((N, D), f32),
        ],
        compiler_params=pltpu.CompilerParams(
            dimension_semantics=("arbitrary",)),
    )(x_risk, money_flow_graph, enc_W, _row2(enc_b), _row2(ln_g), _row2(ln_b),
      att_W1, _row2(att_b1), att_W2, att_b2.reshape(1, 1),
      msg_pos_W, _row2(msg_pos_b), gru_pos_Wih, gru_pos_Whh,
      _row2(gru_pos_bih), _row2(gru_pos_bhh),
      msg_neg_W, _row2(msg_neg_b), gru_neg_Wih, gru_neg_Whh,
      _row2(gru_neg_bih), _row2(gru_neg_bhh),
      comb_W, _row2(comb_b))

    return out[None]


# speedup vs baseline: 1.1255x; 1.0364x over previous
import jax
import jax.numpy as jnp
from jax.experimental import pallas as pl
from jax.experimental.pallas import tpu as pltpu

N = 800
D = 64
F_IN = 128
T_IN = 16
TOP_K = 10
ALPHA = 0.3
RB = 160
BIG_IDX = 1 << 30


def _bdot(a, b, dn=None):
    a = a.astype(jnp.bfloat16).astype(jnp.float32)
    b = b.astype(jnp.bfloat16).astype(jnp.float32)
    if dn is None:
        return jnp.dot(a, b, preferred_element_type=jnp.float32)
    return jax.lax.dot_general(a, b, dn, preferred_element_type=jnp.float32)


_DN_NT = (((1,), (1,)), ((), ()))


def _gru(m, h, Wih, Whh, bih, bhh):
    ir = _bdot(m, Wih[0:D], _DN_NT)
    iz = _bdot(m, Wih[D:2 * D], _DN_NT)
    inn = _bdot(m, Wih[2 * D:], _DN_NT)
    hr = _bdot(h, Whh[0:D], _DN_NT)
    hz = _bdot(h, Whh[D:2 * D], _DN_NT)
    hn = _bdot(h, Whh[2 * D:], _DN_NT)
    r = jax.nn.sigmoid(ir + hr + bih[:, 0:D] + bhh[:, 0:D])
    z = jax.nn.sigmoid(iz + hz + bih[:, D:2 * D] + bhh[:, D:2 * D])
    n = jnp.tanh(inn + bih[:, 2 * D:] + r * (hn + bhh[:, 2 * D:]))
    return (1.0 - z) * n + z * h


def _kernel(x_ref, mfg_ref, encW_ref, encb_ref, lng_ref, lnb_ref,
            w1_ref, b1_ref, w2_ref, b2_ref,
            msgpW_ref, msgpb_ref, pWih_ref, pWhh_ref, pbih_ref, pbhh_ref,
            msgnW_ref, msgnb_ref, nWih_ref, nWhh_ref, nbih_ref, nbhh_ref,
            combW_ref, combb_ref,
            out_ref,
            h_s, sip_s, sjT_s, msgp_s, msgn_s):
    i = pl.program_id(0)
    row0 = i * RB

    @pl.when(i == 0)
    def _prep():
        x = x_ref[0, 0]
        h0 = _bdot(x, encW_ref[...]) + encb_ref[...]
        mu = jnp.mean(h0, axis=1, keepdims=True)
        var = jnp.mean((h0 - mu) ** 2, axis=1, keepdims=True)
        h = jnp.maximum((h0 - mu) / jnp.sqrt(var + 1e-5) * lng_ref[...]
                        + lnb_ref[...], 0.0)
        h_s[...] = h
        sip_s[...] = _bdot(h, w1_ref[0:D]) + b1_ref[...]
        sjT_s[...] = jnp.swapaxes(_bdot(h, w1_ref[D:]), 0, 1)
        msgp_s[...] = _bdot(h, msgpW_ref[...]) + msgpb_ref[...]
        msgn_s[...] = _bdot(h, msgnW_ref[...]) + msgnb_ref[...]

    sip = sip_s[pl.ds(row0, RB), :]
    sjT = sjT_s[...]

    w2q = w2_ref[...].astype(jnp.bfloat16).astype(jnp.float32)
    acc = jnp.zeros((RB, N), dtype=jnp.float32)
    for d in range(D):
        col = sip[:, d:d + 1]
        row = sjT[d:d + 1, :]
        wd = w2q[d:d + 1, :]
        rp = jnp.maximum((col + row).astype(jnp.bfloat16),
                         jnp.bfloat16(0.0)).astype(jnp.float32)
        acc = acc + rp * wd
    logits = acc + b2_ref[...]

    att = jax.nn.sigmoid(logits)
    jota = jax.lax.broadcasted_iota(jnp.int32, (RB, N), 1)
    riota = jax.lax.broadcasted_iota(jnp.int32, (RB, N), 0) + row0
    work = jnp.where(jota == riota, -1.0, att)

    for _ in range(TOP_K):
        mx = jnp.max(work, axis=1, keepdims=True)
        cand = jnp.where(work >= mx, jota, BIG_IDX)
        amin = jnp.min(cand, axis=1, keepdims=True)
        work = jnp.where(jota == amin, -1.0, work)
    mask = ((work == -1.0) & (jota != riota)).astype(jnp.float32)

    att_f = att * mask
    mfg = mfg_ref[0]
    adj_p = att_f * (mfg > ALPHA).astype(jnp.float32)
    adj_p = adj_p / (jnp.sum(adj_p, axis=1, keepdims=True) + 1e-8)
    adj_n = att_f * (mfg < -ALPHA).astype(jnp.float32)
    adj_n = adj_n / (jnp.sum(adj_n, axis=1, keepdims=True) + 1e-8)

    m_pos = _bdot(adj_p, msgp_s[...])
    m_neg = _bdot(adj_n, msgn_s[...])

    h = h_s[pl.ds(row0, RB), :]
    h_pos = _gru(m_pos, h, pWih_ref[...], pWhh_ref[...], pbih_ref[...], pbhh_ref[...])
    h_neg = _gru(m_neg, h, nWih_ref[...], nWhh_ref[...], nbih_ref[...], nbhh_ref[...])

    out_ref[...] = (_bdot(h_pos, combW_ref[0:D]) + _bdot(h_neg, combW_ref[D:])
                    + combb_ref[...])


def _row2(v):
    return v.reshape(1, -1)


def kernel(x_risk, money_flow_graph, enc_W, enc_b, ln_g, ln_b, att_W1, att_b1, att_W2, att_b2,
           msg_pos_W, msg_pos_b, gru_pos_Wih, gru_pos_Whh, gru_pos_bih, gru_pos_bhh,
           msg_neg_W, msg_neg_b, gru_neg_Wih, gru_neg_Whh, gru_neg_bih, gru_neg_bhh,
           comb_W, comb_b):
    f32 = jnp.float32
    grid = N // RB

    full = lambda shape: pl.BlockSpec(shape, lambda i: (0,) * len(shape))
    in_specs = [
        pl.BlockSpec((1, 1, N, F_IN), lambda i: (0, T_IN - 1, 0, 0)),
        pl.BlockSpec((1, RB, N), lambda i: (0, i, 0)),
        full((F_IN, D)),
        full((1, D)),
        full((1, D)),
        full((1, D)),
        full((2 * D, D)),
        full((1, D)),
        full((D, 1)),
        full((1, 1)),
        full((D, D)),
        full((1, D)),
        full((3 * D, D)),
        full((3 * D, D)),
        full((1, 3 * D)),
        full((1, 3 * D)),
        full((D, D)),
        full((1, D)),
        full((3 * D, D)),
        full((3 * D, D)),
        full((1, 3 * D)),
        full((1, 3 * D)),
        full((2 * D, D)),
        full((1, D)),
    ]

    out = pl.pallas_call(
        _kernel,
        grid=(grid,),
        in_specs=in_specs,
        out_specs=pl.BlockSpec((RB, D), lambda i: (i, 0)),
        out_shape=jax.ShapeDtypeStruct((N, D), f32),
        scratch_shapes=[
            pltpu.VMEM((N, D), f32),
            pltpu.VMEM((N, D), f32),
            pltpu.VMEM((D, N), f32),
            pltpu.VMEM((N, D), f32),
            pltpu.VMEM((N, D), f32),
        ],
        compiler_params=pltpu.CompilerParams(
            dimension_semantics=("arbitrary",)),
    )(x_risk, money_flow_graph, enc_W, _row2(enc_b), _row2(ln_g), _row2(ln_b),
      att_W1, _row2(att_b1), att_W2, att_b2.reshape(1, 1),
      msg_pos_W, _row2(msg_pos_b), gru_pos_Wih, gru_pos_Whh,
      _row2(gru_pos_bih), _row2(gru_pos_bhh),
      msg_neg_W, _row2(msg_neg_b), gru_neg_Wih, gru_neg_Whh,
      _row2(gru_neg_bih), _row2(gru_neg_bhh),
      comb_W, _row2(comb_b))

    return out[None]


# speedup vs baseline: 1.2370x; 1.0991x over previous
import jax
import jax.numpy as jnp
from jax.experimental import pallas as pl
from jax.experimental.pallas import tpu as pltpu

N = 800
D = 64
F_IN = 128
T_IN = 16
TOP_K = 10
ALPHA = 0.3
RB = 400
BIG_IDX = 1 << 30


def _bdot(a, b, dn=None):
    a = a.astype(jnp.bfloat16).astype(jnp.float32)
    b = b.astype(jnp.bfloat16).astype(jnp.float32)
    if dn is None:
        return jnp.dot(a, b, preferred_element_type=jnp.float32)
    return jax.lax.dot_general(a, b, dn, preferred_element_type=jnp.float32)


_DN_NT = (((1,), (1,)), ((), ()))


def _gru(m, h, Wih, Whh, bih, bhh):
    ir = _bdot(m, Wih[0:D], _DN_NT)
    iz = _bdot(m, Wih[D:2 * D], _DN_NT)
    inn = _bdot(m, Wih[2 * D:], _DN_NT)
    hr = _bdot(h, Whh[0:D], _DN_NT)
    hz = _bdot(h, Whh[D:2 * D], _DN_NT)
    hn = _bdot(h, Whh[2 * D:], _DN_NT)
    r = jax.nn.sigmoid(ir + hr + bih[:, 0:D] + bhh[:, 0:D])
    z = jax.nn.sigmoid(iz + hz + bih[:, D:2 * D] + bhh[:, D:2 * D])
    n = jnp.tanh(inn + bih[:, 2 * D:] + r * (hn + bhh[:, 2 * D:]))
    return (1.0 - z) * n + z * h


def _kernel(x_ref, mfg_ref, encW_ref, encb_ref, lng_ref, lnb_ref,
            w1_ref, b1_ref, w2_ref, b2_ref,
            msgpW_ref, msgpb_ref, pWih_ref, pWhh_ref, pbih_ref, pbhh_ref,
            msgnW_ref, msgnb_ref, nWih_ref, nWhh_ref, nbih_ref, nbhh_ref,
            combW_ref, combb_ref,
            out_ref,
            h_s, sip_s, sjT_s, msgp_s, msgn_s):
    i = pl.program_id(0)
    row0 = i * RB

    @pl.when(i == 0)
    def _prep():
        x = x_ref[0, 0]
        h0 = _bdot(x, encW_ref[...]) + encb_ref[...]
        mu = jnp.mean(h0, axis=1, keepdims=True)
        var = jnp.mean((h0 - mu) ** 2, axis=1, keepdims=True)
        h = jnp.maximum((h0 - mu) / jnp.sqrt(var + 1e-5) * lng_ref[...]
                        + lnb_ref[...], 0.0)
        h_s[...] = h
        sip_s[...] = _bdot(h, w1_ref[0:D]) + b1_ref[...]
        sjT_s[...] = jnp.swapaxes(_bdot(h, w1_ref[D:]), 0, 1)
        msgp_s[...] = _bdot(h, msgpW_ref[...]) + msgpb_ref[...]
        msgn_s[...] = _bdot(h, msgnW_ref[...]) + msgnb_ref[...]

    sip = sip_s[pl.ds(row0, RB), :]
    sjT = sjT_s[...]

    w2q = w2_ref[...].astype(jnp.bfloat16).astype(jnp.float32)
    acc = jnp.zeros((RB, N), dtype=jnp.float32)
    for d in range(D):
        col = sip[:, d:d + 1]
        row = sjT[d:d + 1, :]
        wd = w2q[d:d + 1, :]
        rp = jnp.maximum((col + row).astype(jnp.bfloat16),
                         jnp.bfloat16(0.0)).astype(jnp.float32)
        acc = acc + rp * wd
    logits = acc + b2_ref[...]

    att = jax.nn.sigmoid(logits)
    jota = jax.lax.broadcasted_iota(jnp.int32, (RB, N), 1)
    riota = jax.lax.broadcasted_iota(jnp.int32, (RB, N), 0) + row0
    work = jnp.where(jota == riota, -1.0, att)

    for _ in range(TOP_K):
        mx = jnp.max(work, axis=1, keepdims=True)
        cand = jnp.where(work >= mx, jota, BIG_IDX)
        amin = jnp.min(cand, axis=1, keepdims=True)
        work = jnp.where(jota == amin, -1.0, work)
    mask = ((work == -1.0) & (jota != riota)).astype(jnp.float32)

    att_f = att * mask
    mfg = mfg_ref[0]
    adj_p = att_f * (mfg > ALPHA).astype(jnp.float32)
    adj_p = adj_p / (jnp.sum(adj_p, axis=1, keepdims=True) + 1e-8)
    adj_n = att_f * (mfg < -ALPHA).astype(jnp.float32)
    adj_n = adj_n / (jnp.sum(adj_n, axis=1, keepdims=True) + 1e-8)

    m_pos = _bdot(adj_p, msgp_s[...])
    m_neg = _bdot(adj_n, msgn_s[...])

    h = h_s[pl.ds(row0, RB), :]
    h_pos = _gru(m_pos, h, pWih_ref[...], pWhh_ref[...], pbih_ref[...], pbhh_ref[...])
    h_neg = _gru(m_neg, h, nWih_ref[...], nWhh_ref[...], nbih_ref[...], nbhh_ref[...])

    out_ref[...] = (_bdot(h_pos, combW_ref[0:D]) + _bdot(h_neg, combW_ref[D:])
                    + combb_ref[...])


def _row2(v):
    return v.reshape(1, -1)


def kernel(x_risk, money_flow_graph, enc_W, enc_b, ln_g, ln_b, att_W1, att_b1, att_W2, att_b2,
           msg_pos_W, msg_pos_b, gru_pos_Wih, gru_pos_Whh, gru_pos_bih, gru_pos_bhh,
           msg_neg_W, msg_neg_b, gru_neg_Wih, gru_neg_Whh, gru_neg_bih, gru_neg_bhh,
           comb_W, comb_b):
    f32 = jnp.float32
    grid = N // RB

    full = lambda shape: pl.BlockSpec(shape, lambda i: (0,) * len(shape))
    in_specs = [
        pl.BlockSpec((1, 1, N, F_IN), lambda i: (0, T_IN - 1, 0, 0)),
        pl.BlockSpec((1, RB, N), lambda i: (0, i, 0)),
        full((F_IN, D)),
        full((1, D)),
        full((1, D)),
        full((1, D)),
        full((2 * D, D)),
        full((1, D)),
        full((D, 1)),
        full((1, 1)),
        full((D, D)),
        full((1, D)),
        full((3 * D, D)),
        full((3 * D, D)),
        full((1, 3 * D)),
        full((1, 3 * D)),
        full((D, D)),
        full((1, D)),
        full((3 * D, D)),
        full((3 * D, D)),
        full((1, 3 * D)),
        full((1, 3 * D)),
        full((2 * D, D)),
        full((1, D)),
    ]

    out = pl.pallas_call(
        _kernel,
        grid=(grid,),
        in_specs=in_specs,
        out_specs=pl.BlockSpec((RB, D), lambda i: (i, 0)),
        out_shape=jax.ShapeDtypeStruct((N, D), f32),
        scratch_shapes=[
            pltpu.VMEM((N, D), f32),
            pltpu.VMEM((N, D), f32),
            pltpu.VMEM((D, N), f32),
            pltpu.VMEM((N, D), f32),
            pltpu.VMEM((N, D), f32),
        ],
        compiler_params=pltpu.CompilerParams(
            dimension_semantics=("arbitrary",)),
    )(x_risk, money_flow_graph, enc_W, _row2(enc_b), _row2(ln_g), _row2(ln_b),
      att_W1, _row2(att_b1), att_W2, att_b2.reshape(1, 1),
      msg_pos_W, _row2(msg_pos_b), gru_pos_Wih, gru_pos_Whh,
      _row2(gru_pos_bih), _row2(gru_pos_bhh),
      msg_neg_W, _row2(msg_neg_b), gru_neg_Wih, gru_neg_Whh,
      _row2(gru_neg_bih), _row2(gru_neg_bhh),
      comb_W, _row2(comb_b))

    return out[None]
